# Initial kernel scaffold; baseline (speedup 1.0000x reference)
import jax
import jax.numpy as jnp
from jax.experimental import pallas as pl


def kernel(x, edge_index, batch, W1, b1, W2, b2, W3, b3, lw1, lb1, lw2, lb2, lw3, lb3):
    raise NotImplementedError("write your pallas kernel here")


# speedup vs baseline: 8.9779x; 8.9779x over previous
import functools

import jax
import jax.numpy as jnp
from jax import lax
from jax.experimental import pallas as pl
from jax.experimental.pallas import tpu as pltpu
from jax.experimental.pallas import tpu_sc as plsc

N = 10000
E = 320000
DF = 128
NH = 256
G = 16
NC = 2
NS = 16
EK = 80
RW = 632


@functools.cache
def _get_mesh():
    return plsc.VectorSubcoreMesh(
        core_axis_name="c", subcore_axis_name="s",
        num_cores=NC, num_subcores=NS)


def _row_base(sid):
    return jnp.minimum(sid * RW, N - RW)


@functools.cache
def _get_sc_deg():
    epw = E // (NC * NS)
    nch = epw // EK

    @functools.partial(
        pl.kernel,
        out_type=jax.ShapeDtypeStruct((2 * N, 128), jnp.float32),
        mesh=_get_mesh(),
        scratch_types=[
            pltpu.VMEM_SHARED((N, 128), jnp.float32),
            pltpu.VMEM((EK,), jnp.int32),
            pltpu.VMEM((EK, 128), jnp.float32),
        ],
    )
    def _sc_deg(dst_hbm, z_hbm, o_hbm, deg_hbm, acc, didx, ones_v):
        cid = lax.axis_index("c")
        sid = lax.axis_index("s")
        r0 = _row_base(sid)
        pltpu.sync_copy(z_hbm, acc.at[pl.ds(r0, RW)])
        pltpu.sync_copy(o_hbm, ones_v)
        plsc.subcore_barrier()
        base = (cid * NS + sid) * epw

        def step(c, carry):
            pltpu.sync_copy(dst_hbm.at[pl.ds(base + c * EK, EK)], didx)
            pltpu.sync_copy(ones_v, acc.at[didx], add=True)
            return carry

        lax.fori_loop(0, nch, step, 0)
        plsc.subcore_barrier()
        pltpu.sync_copy(acc.at[pl.ds(r0, RW)],
                        deg_hbm.at[pl.ds(cid * N + r0, RW)])

    return _sc_deg


@functools.cache
def _get_sc_scatter(split_edges):
    epw = E // (NC * NS) if split_edges else E // NS
    nch = epw // EK
    ny = N if split_edges else 2 * N

    @functools.partial(
        pl.kernel,
        out_type=jax.ShapeDtypeStruct((2 * N, 128), jnp.float32),
        mesh=_get_mesh(),
        scratch_types=[
            pltpu.VMEM_SHARED((N, 128), jnp.float32),
            pltpu.VMEM((EK,), jnp.int32),
            pltpu.VMEM((EK,), jnp.int32),
            pltpu.VMEM((EK, 128), jnp.float32),
            pltpu.SemaphoreType.DMA,
        ],
    )
    def _sc_scatter(src_hbm, dst_hbm, y_hbm, z_hbm, s_hbm,
                    acc, sidx, didx, rows, gsem):
        cid = lax.axis_index("c")
        sid = lax.axis_index("s")
        r0 = _row_base(sid)
        pltpu.sync_copy(z_hbm, acc.at[pl.ds(r0, RW)])
        plsc.subcore_barrier()
        if split_edges:
            base = (cid * NS + sid) * epw
        else:
            base = sid * epw
        delta = cid * jnp.int32(0 if split_edges else N)

        def step(c, carry):
            off = base + c * EK
            pltpu.sync_copy(src_hbm.at[pl.ds(off, EK)], sidx)
            pltpu.sync_copy(dst_hbm.at[pl.ds(off, EK)], didx)
            if not split_edges:
                for j in range(EK // 16):
                    sl = pl.ds(j * 16, 16)
                    sidx[sl] = sidx[sl] + delta
            pltpu.async_copy(y_hbm.at[sidx], rows, gsem).wait()
            pltpu.sync_copy(rows, acc.at[didx], add=True)
            return carry

        lax.fori_loop(0, nch, step, 0)
        plsc.subcore_barrier()
        pltpu.sync_copy(acc.at[pl.ds(r0, RW)],
                        s_hbm.at[pl.ds(cid * N + r0, RW)])

    del ny
    return _sc_scatter


R = 1000
GRID = N // R


def _tc_prep_body(deg0, deg1, xb, batchb, dinv_ref, y_ref, cnt_ref):
    i = pl.program_id(0)
    deg = deg0[:, 0:1] + deg1[:, 0:1] + 1.0
    dinv = lax.rsqrt(deg)
    dinv_ref[...] = dinv
    y_ref[...] = dinv * xb[...]
    b = batchb[...]
    oh = (b == lax.broadcasted_iota(jnp.int32, (1, G), 1)).astype(jnp.float32)

    @pl.when(i == 0)
    def _():
        cnt_ref[...] = jnp.zeros_like(cnt_ref)

    cnt_ref[...] += lax.dot_general(
        oh, jnp.ones((R, 128), jnp.float32),
        (((0,), (0,)), ((), ())), preferred_element_type=jnp.float32)


_tc_prep = pl.pallas_call(
    _tc_prep_body,
    grid=(GRID,),
    in_specs=[
        pl.BlockSpec((R, 128), lambda i: (i, 0)),
        pl.BlockSpec((R, 128), lambda i: (GRID + i, 0)),
        pl.BlockSpec((R, DF), lambda i: (i, 0)),
        pl.BlockSpec((R, 1), lambda i: (i, 0)),
    ],
    out_specs=[
        pl.BlockSpec((R, 1), lambda i: (i, 0)),
        pl.BlockSpec((R, DF), lambda i: (i, 0)),
        pl.BlockSpec((G, 128), lambda i: (0, 0)),
    ],
    out_shape=[
        jax.ShapeDtypeStruct((N, 1), jnp.float32),
        jax.ShapeDtypeStruct((N, DF), jnp.float32),
        jax.ShapeDtypeStruct((G, 128), jnp.float32),
    ],
)


def _pool_update(i, b, hb, gmax_ref, gsum_ref):
    oh = (b == lax.broadcasted_iota(jnp.int32, (1, G), 1)).astype(jnp.float32)
    gs = lax.dot_general(oh, hb, (((0,), (0,)), ((), ())),
                         preferred_element_type=jnp.float32)

    @pl.when(i == 0)
    def _():
        gmax_ref[...] = jnp.zeros_like(gmax_ref)
        gsum_ref[...] = jnp.zeros_like(gsum_ref)

    gsum_ref[...] += gs
    for g in range(G):
        vals = jnp.where(b == g, hb, 0.0)
        bm = jnp.max(vals, axis=0, keepdims=True)
        gmax_ref[pl.ds(g, 1), :] = jnp.maximum(gmax_ref[pl.ds(g, 1), :], bm)


def _agg_block(s0, s1, hp, dv, combine):
    if combine == "add":
        s = s0[...] + s1[...]
    else:
        s = jnp.concatenate([s0[...], s1[...]], axis=1)
    return dv * s + (dv * dv) * hp[...]


def _make_tc_layer(din, combine):
    def body(s0, s1, hp, dinv, batchb, W, bb,
             h_ref, yn_ref, gmax_ref, gsum_ref):
        i = pl.program_id(0)
        dv = dinv[...]
        agg = _agg_block(s0, s1, hp, dv, combine)
        hb = jnp.maximum(
            jnp.dot(agg, W[...], preferred_element_type=jnp.float32)
            + bb[...], 0.0)
        h_ref[...] = hb
        yn = dv * hb
        yn_ref[0] = yn[:, :NH // 2]
        yn_ref[1] = yn[:, NH // 2:]
        _pool_update(i, batchb[...], hb, gmax_ref, gsum_ref)

    return pl.pallas_call(
        body,
        grid=(GRID,),
        in_specs=[
            pl.BlockSpec((R, 128), lambda i: (i, 0)),
            pl.BlockSpec((R, 128), lambda i: (GRID + i, 0)),
            pl.BlockSpec((R, din), lambda i: (i, 0)),
            pl.BlockSpec((R, 1), lambda i: (i, 0)),
            pl.BlockSpec((R, 1), lambda i: (i, 0)),
            pl.BlockSpec((din, NH), lambda i: (0, 0)),
            pl.BlockSpec((1, NH), lambda i: (0, 0)),
        ],
        out_specs=[
            pl.BlockSpec((R, NH), lambda i: (i, 0)),
            pl.BlockSpec((2, R, NH // 2), lambda i: (0, i, 0)),
            pl.BlockSpec((G, NH), lambda i: (0, 0)),
            pl.BlockSpec((G, NH), lambda i: (0, 0)),
        ],
        out_shape=[
            jax.ShapeDtypeStruct((N, NH), jnp.float32),
            jax.ShapeDtypeStruct((2, N, NH // 2), jnp.float32),
            jax.ShapeDtypeStruct((G, NH), jnp.float32),
            jax.ShapeDtypeStruct((G, NH), jnp.float32),
        ],
    )


_tc_layer1 = _make_tc_layer(DF, "add")
_tc_layer2 = _make_tc_layer(NH, "concat")


def _tc_final_body(s0, s1, hp, dinv, batchb, W, bb,
                   gmax1, gsum1, gmax2, gsum2, cnt,
                   lw1, lb1, lw2, lb2, lw3, lb3,
                   out_ref, gmax_ref, gsum_ref):
    i = pl.program_id(0)
    dv = dinv[...]
    agg = _agg_block(s0, s1, hp, dv, "concat")
    hb = jnp.maximum(
        jnp.dot(agg, W[...], preferred_element_type=jnp.float32) + bb[...],
        0.0)
    _pool_update(i, batchb[...], hb, gmax_ref, gsum_ref)

    @pl.when(i == GRID - 1)
    def _():
        c = jnp.maximum(cnt[...][:, 0:1], 1.0)
        xo = (jnp.maximum(jnp.concatenate(
                  [gmax1[...], gsum1[...] / c], axis=1), 0.0)
              + jnp.maximum(jnp.concatenate(
                  [gmax2[...], gsum2[...] / c], axis=1), 0.0)
              + jnp.maximum(jnp.concatenate(
                  [gmax_ref[...], gsum_ref[...] / c], axis=1), 0.0))
        o = jnp.maximum(
            jnp.dot(xo, lw1[...], preferred_element_type=jnp.float32)
            + lb1[...], 0.0)
        o = jnp.maximum(
            jnp.dot(o, lw2[...], preferred_element_type=jnp.float32)
            + lb2[...], 0.0)
        o = (jnp.dot(o, lw3[...], preferred_element_type=jnp.float32)
             + lb3[...])
        m = jnp.max(o, axis=-1, keepdims=True)
        z = o - m
        out_ref[...] = z - jnp.log(jnp.sum(jnp.exp(z), axis=-1,
                                           keepdims=True))


_tc_final = pl.pallas_call(
    _tc_final_body,
    grid=(GRID,),
    in_specs=[
        pl.BlockSpec((R, 128), lambda i: (i, 0)),
        pl.BlockSpec((R, 128), lambda i: (GRID + i, 0)),
        pl.BlockSpec((R, NH), lambda i: (i, 0)),
        pl.BlockSpec((R, 1), lambda i: (i, 0)),
        pl.BlockSpec((R, 1), lambda i: (i, 0)),
        pl.BlockSpec((NH, NH), lambda i: (0, 0)),
        pl.BlockSpec((1, NH), lambda i: (0, 0)),
        pl.BlockSpec((G, NH), lambda i: (0, 0)),
        pl.BlockSpec((G, NH), lambda i: (0, 0)),
        pl.BlockSpec((G, NH), lambda i: (0, 0)),
        pl.BlockSpec((G, NH), lambda i: (0, 0)),
        pl.BlockSpec((G, 128), lambda i: (0, 0)),
        pl.BlockSpec((2 * NH, NH), lambda i: (0, 0)),
        pl.BlockSpec((1, NH), lambda i: (0, 0)),
        pl.BlockSpec((NH, NH // 2), lambda i: (0, 0)),
        pl.BlockSpec((1, NH // 2), lambda i: (0, 0)),
        pl.BlockSpec((NH // 2, 10), lambda i: (0, 0)),
        pl.BlockSpec((1, 10), lambda i: (0, 0)),
    ],
    out_specs=[
        pl.BlockSpec((G, 10), lambda i: (0, 0)),
        pl.BlockSpec((G, NH), lambda i: (0, 0)),
        pl.BlockSpec((G, NH), lambda i: (0, 0)),
    ],
    out_shape=[
        jax.ShapeDtypeStruct((G, 10), jnp.float32),
        jax.ShapeDtypeStruct((G, NH), jnp.float32),
        jax.ShapeDtypeStruct((G, NH), jnp.float32),
    ],
)


def kernel(x, edge_index, batch, W1, b1, W2, b2, W3, b3,
           lw1, lb1, lw2, lb2, lw3, lb3):
    src = edge_index[0].astype(jnp.int32)
    dst = edge_index[1].astype(jnp.int32)
    batch2 = batch.astype(jnp.int32)[:, None]
    z128 = jnp.zeros((RW, 128), jnp.float32)
    o128 = jnp.ones((EK, 128), jnp.float32)

    degp = _get_sc_deg()(dst, z128, o128)
    dinv, y, cnt = _tc_prep(degp, degp, x, batch2)

    s = _get_sc_scatter(True)(src, dst, y, z128)
    h1, yn, gmax1, gsum1 = _tc_layer1(
        s, s, x, dinv, batch2, W1, b1[None, :])
    yn = yn.reshape(2 * N, NH // 2)

    s = _get_sc_scatter(False)(src, dst, yn, z128)
    h2, yn, gmax2, gsum2 = _tc_layer2(
        s, s, h1, dinv, batch2, W2, b2[None, :])
    yn = yn.reshape(2 * N, NH // 2)

    s = _get_sc_scatter(False)(src, dst, yn, z128)
    out, _, _ = _tc_final(
        s, s, h2, dinv, batch2, W3, b3[None, :],
        gmax1, gsum1, gmax2, gsum2, cnt,
        lw1, lb1[None, :], lw2, lb2[None, :], lw3, lb3[None, :])
    return out
